# baseline (speedup 1.0000x reference)
import functools

import jax
import jax.numpy as jnp
from jax.experimental import pallas as pl
from jax.experimental.pallas import tpu as pltpu

_KNN = 20
_TOP = _KNN + 1
_PAD_VAL = 1e10
_SUB = 128


def _lid_body(nc, qt, kc, jsel, q_ref, kt_ref, out_ref, d2_ref, cand_ref,
              kn_ref):
    nsub = kc // _SUB
    nchunks = nc * nsub
    ncand = jsel * _SUB

    q = q_ref[...]
    qn = jnp.sum(q * q, axis=1, keepdims=True)[:, :, None]

    @pl.when(pl.program_id(0) == 0)
    def _():
        for c in range(nc):
            kt = kt_ref[c]
            kn_ref[:, c * nsub:(c + 1) * nsub, :] = jnp.broadcast_to(
                jnp.sum(kt * kt, axis=0, keepdims=True), (8, nsub, _SUB))

    m_parts = []
    for c in range(nc):
        kt = kt_ref[c]
        kn = kn_ref[0:1, c * nsub:(c + 1) * nsub, :]
        qk = jax.lax.dot_general(q, kt, (((1,), (0,)), ((), ())),
                                 preferred_element_type=jnp.float32)
        d3 = jnp.maximum(qn + kn - 2.0 * qk, 1e-12)
        d2_ref[:, c * nsub:(c + 1) * nsub, :] = d3
        m_parts.append(jnp.min(d3, axis=2))
    minima = jnp.concatenate(m_parts, axis=1)

    ciota = jax.lax.broadcasted_iota(jnp.int32, (qt, nchunks), 1)
    jiota = jax.lax.broadcasted_iota(jnp.int32, (qt, jsel), 1)

    def extract(j, carry):
        mins, idxs = carry
        vmin = jnp.min(mins, axis=1, keepdims=True)
        imin = jnp.min(jnp.where(mins == vmin, ciota, jnp.int32(1 << 30)),
                       axis=1, keepdims=True)
        mins = jnp.where(ciota == imin, jnp.float32(3e38), mins)
        idxs = jnp.where(jiota == j, imin, idxs)
        return mins, idxs

    _, idxs = jax.lax.fori_loop(
        0, jsel, extract, (minima, jnp.zeros((qt, jsel), jnp.int32)))

    mask16 = jnp.int32(-65536)
    ciota3 = jax.lax.broadcasted_iota(jnp.int32, (qt, jsel, nchunks), 2)
    g = (idxs[:, :, None] == ciota3).astype(jnp.float32)
    dn = (((2,), (1,)), ((0,), (0,)))
    nblk = 6 if (nchunks % 6 == 0 and (nchunks // 6) % 8 == 0) else 1
    bw = nchunks // nblk
    z = jnp.zeros((qt, jsel, _SUB), jnp.float32)
    for b in range(nblk):
        db = d2_ref[:, b * bw:(b + 1) * bw, :]
        h0 = jax.lax.bitcast_convert_type(
            jax.lax.bitcast_convert_type(db, jnp.int32) & mask16,
            jnp.float32)
        h1 = db - h0
        gb = g[:, :, b * bw:(b + 1) * bw]
        z = (z + jax.lax.dot_general(gb, h0, dn,
                                     preferred_element_type=jnp.float32)
             + jax.lax.dot_general(gb, h1, dn,
                                   preferred_element_type=jnp.float32))
    cand_ref[...] = z.reshape(qt, ncand)

    cand = cand_ref[...]
    lo_f = jnp.min(cand, axis=1, keepdims=True)
    hi_f = jnp.max(cand, axis=1, keepdims=True)
    lo = jax.lax.bitcast_convert_type(lo_f, jnp.int32)
    hi = jax.lax.bitcast_convert_type(hi_f, jnp.int32)

    def bisect_step(_, carry):
        lo, hi = carry
        mid = lo + (hi - lo) // 2
        midf = jax.lax.bitcast_convert_type(mid, jnp.float32)
        cnt = jnp.sum((cand_ref[...] <= midf).astype(jnp.int32),
                      axis=1, keepdims=True)
        pred = cnt >= _TOP
        return jnp.where(pred, lo, mid + 1), jnp.where(pred, mid, hi)

    lo, _ = jax.lax.fori_loop(0, 31, bisect_step, (lo, hi))
    r2 = jax.lax.bitcast_convert_type(lo, jnp.float32)

    r = jnp.sqrt(r2)
    inv = 1.0 / (r + 1e-12)
    cand = cand_ref[...]
    term = jnp.log(jnp.sqrt(cand) * inv + 1e-12)
    strict = cand < r2
    m = jnp.sum(strict.astype(jnp.float32), axis=1, keepdims=True)
    sl = jnp.sum(jnp.where(strict, term, 0.0), axis=1, keepdims=True)
    tie = jnp.log(r * inv + 1e-12)
    nearest = jnp.log(jnp.sqrt(lo_f) * inv + 1e-12)
    logsum = sl + (float(_TOP) - m) * tie - nearest
    out_ref[...] = -(float(_KNN) / logsum)


def kernel(queries, keys):
    qtotal, dim = queries.shape
    ktotal = keys.shape[0]

    qt = 64 if qtotal % 64 == 0 else qtotal
    kc = 6144
    if ktotal <= kc:
        kc = ((ktotal + _SUB - 1) // _SUB) * _SUB
    nc = (ktotal + kc - 1) // kc
    kpad = nc * kc
    jsel = min(_TOP, (nc * kc) // _SUB)

    keys_t = jnp.pad(keys.T, ((0, 0), (0, kpad - ktotal)),
                     constant_values=_PAD_VAL)
    keys_t3 = keys_t.reshape(dim, nc, kc).transpose(1, 0, 2)
    keys_t3 = keys_t3.reshape(nc, dim, kc // _SUB, _SUB)

    body = functools.partial(_lid_body, nc, qt, kc, jsel)
    out = pl.pallas_call(
        body,
        grid=(qtotal // qt,),
        in_specs=[
            pl.BlockSpec((qt, dim), lambda i: (i, 0)),
            pl.BlockSpec((nc, dim, kc // _SUB, _SUB),
                         lambda i: (0, 0, 0, 0)),
        ],
        out_specs=pl.BlockSpec((qt, 1), lambda i: (i, 0)),
        out_shape=jax.ShapeDtypeStruct((qtotal, 1), jnp.float32),
        scratch_shapes=[
            pltpu.VMEM((qt, (nc * kc) // _SUB, _SUB), jnp.float32),
            pltpu.VMEM((qt, jsel * _SUB), jnp.float32),
            pltpu.VMEM((8, (nc * kc) // _SUB, _SUB), jnp.float32),
        ],
    )(queries, keys_t3)
    return out


# speedup vs baseline: 1.0218x; 1.0218x over previous
import functools

import jax
import jax.numpy as jnp
from jax.experimental import pallas as pl
from jax.experimental.pallas import tpu as pltpu

_KNN = 20
_TOP = _KNN + 1
_PAD_VAL = 1e10
_SUB = 128


def _lid_body(nc, qt, kc, jsel, q_ref, kt_ref, out_ref, d2_ref, cand_ref):
    nsub = kc // _SUB
    nchunks = nc * nsub
    ncand = jsel * _SUB

    q = q_ref[...]
    qn = jnp.sum(q * q, axis=1, keepdims=True)

    m_parts = []
    for c in range(nc):
        kt = kt_ref[c]
        kn = jnp.sum(kt * kt, axis=0, keepdims=True)
        qk = jax.lax.dot_general(q, kt, (((1,), (0,)), ((), ())),
                                 preferred_element_type=jnp.float32)
        d2 = jnp.maximum(qn + kn - 2.0 * qk, 1e-12)
        d3 = d2.reshape(qt, nsub, _SUB)
        d2_ref[:, c * nsub:(c + 1) * nsub, :] = d3
        m_parts.append(jnp.min(d3, axis=2))
    minima = jnp.concatenate(m_parts, axis=1)

    ciota = jax.lax.broadcasted_iota(jnp.int32, (qt, nchunks), 1)
    jiota = jax.lax.broadcasted_iota(jnp.int32, (qt, jsel), 1)

    def extract(j, carry):
        mins, idxs = carry
        vmin = jnp.min(mins, axis=1, keepdims=True)
        imin = jnp.min(jnp.where(mins == vmin, ciota, jnp.int32(1 << 30)),
                       axis=1, keepdims=True)
        mins = jnp.where(ciota == imin, jnp.float32(3e38), mins)
        idxs = jnp.where(jiota == j, imin, idxs)
        return mins, idxs

    _, idxs = jax.lax.fori_loop(
        0, jsel, extract, (minima, jnp.zeros((qt, jsel), jnp.int32)))

    mask16 = jnp.int32(-65536)
    ciota3 = jax.lax.broadcasted_iota(jnp.int32, (qt, jsel, nchunks), 2)
    g = (idxs[:, :, None] == ciota3).astype(jnp.float32)
    dn = (((2,), (1,)), ((0,), (0,)))
    nblk = 6 if (nchunks % 6 == 0 and (nchunks // 6) % 8 == 0) else 1
    bw = nchunks // nblk
    z = jnp.zeros((qt, jsel, _SUB), jnp.float32)
    for b in range(nblk):
        db = d2_ref[:, b * bw:(b + 1) * bw, :]
        h0 = jax.lax.bitcast_convert_type(
            jax.lax.bitcast_convert_type(db, jnp.int32) & mask16,
            jnp.float32)
        h1 = db - h0
        gb = g[:, :, b * bw:(b + 1) * bw]
        z = (z + jax.lax.dot_general(gb, h0, dn,
                                     preferred_element_type=jnp.float32)
             + jax.lax.dot_general(gb, h1, dn,
                                   preferred_element_type=jnp.float32))
    cand_ref[...] = z.reshape(qt, ncand)

    cand = cand_ref[...]
    lo_f = jnp.min(cand, axis=1, keepdims=True)
    hi_f = jnp.max(cand, axis=1, keepdims=True)
    lo = jax.lax.bitcast_convert_type(lo_f, jnp.int32)
    hi = jax.lax.bitcast_convert_type(hi_f, jnp.int32)

    def bisect_step(_, carry):
        lo, hi = carry
        mid = lo + (hi - lo) // 2
        midf = jax.lax.bitcast_convert_type(mid, jnp.float32)
        cnt = jnp.sum((cand_ref[...] <= midf).astype(jnp.int32),
                      axis=1, keepdims=True)
        pred = cnt >= _TOP
        return jnp.where(pred, lo, mid + 1), jnp.where(pred, mid, hi)

    lo, _ = jax.lax.fori_loop(0, 31, bisect_step, (lo, hi))
    r2 = jax.lax.bitcast_convert_type(lo, jnp.float32)

    r = jnp.sqrt(r2)
    inv = 1.0 / (r + 1e-12)
    cand = cand_ref[...]
    term = jnp.log(jnp.sqrt(cand) * inv + 1e-12)
    strict = cand < r2
    m = jnp.sum(strict.astype(jnp.float32), axis=1, keepdims=True)
    sl = jnp.sum(jnp.where(strict, term, 0.0), axis=1, keepdims=True)
    tie = jnp.log(r * inv + 1e-12)
    nearest = jnp.log(jnp.sqrt(lo_f) * inv + 1e-12)
    logsum = sl + (float(_TOP) - m) * tie - nearest
    out_ref[...] = -(float(_KNN) / logsum)


def kernel(queries, keys):
    qtotal, dim = queries.shape
    ktotal = keys.shape[0]

    qt = 64 if qtotal % 64 == 0 else qtotal
    kc = 6144
    if ktotal <= kc:
        kc = ((ktotal + _SUB - 1) // _SUB) * _SUB
    nc = (ktotal + kc - 1) // kc
    kpad = nc * kc
    jsel = min(_TOP, (nc * kc) // _SUB)

    keys_t = jnp.pad(keys.T, ((0, 0), (0, kpad - ktotal)),
                     constant_values=_PAD_VAL)
    keys_t3 = keys_t.reshape(dim, nc, kc).transpose(1, 0, 2)

    body = functools.partial(_lid_body, nc, qt, kc, jsel)
    out = pl.pallas_call(
        body,
        grid=(qtotal // qt,),
        in_specs=[
            pl.BlockSpec((qt, dim), lambda i: (i, 0)),
            pl.BlockSpec((nc, dim, kc), lambda i: (0, 0, 0)),
        ],
        out_specs=pl.BlockSpec((qt, 1), lambda i: (i, 0)),
        out_shape=jax.ShapeDtypeStruct((qtotal, 1), jnp.float32),
        scratch_shapes=[
            pltpu.VMEM((qt, (nc * kc) // _SUB, _SUB), jnp.float32),
            pltpu.VMEM((qt, jsel * _SUB), jnp.float32),
        ],
    )(queries, keys_t3)
    return out
